# baseline (speedup 1.0000x reference)
import functools

import jax
import jax.numpy as jnp
from jax import lax
from jax.experimental import pallas as pl
from jax.experimental.pallas import tpu as pltpu
from jax.experimental.pallas import tpu_sc as plsc

_NUM_CORES = 2
_NUM_SUBCORES = 16
_NW = _NUM_CORES * _NUM_SUBCORES
_LANES = 16


@functools.lru_cache(maxsize=None)
def _make_kernel(B, T, D, V):
    assert T % _NW == 0 and D % _LANES == 0
    tpw = T // _NW
    half = tpw // 2
    groups = D // _LANES

    mesh = plsc.VectorSubcoreMesh(core_axis_name="c", subcore_axis_name="s")

    @functools.partial(
        pl.kernel,
        mesh=mesh,
        out_type=jax.ShapeDtypeStruct((B * T, D), jnp.float32),
        scratch_types=[
            pltpu.VMEM((B, tpw), jnp.int32),
            pltpu.VMEM((tpw, D), jnp.float32),
            pltpu.VMEM((tpw, D), jnp.float32),
            pltpu.SemaphoreType.DMA,
            pltpu.SemaphoreType.DMA,
            pltpu.SemaphoreType.DMA,
        ],
    )
    def emb(x_hbm, tok_hbm, pos_hbm, out_hbm, idx_v, rows_v, pos_v,
            sem_i, sem_g, sem_w):
        wid = lax.axis_index("s") * _NUM_CORES + lax.axis_index("c")
        t0 = wid * tpw

        idx_d = [pltpu.async_copy(x_hbm.at[b, pl.ds(t0, tpw)],
                                  idx_v.at[b], sem_i) for b in range(B)]
        pos_d = pltpu.async_copy(pos_hbm.at[pl.ds(t0, tpw)], pos_v, sem_g)

        def add_rows(r_lo, r_hi):
            def row_add(r, carry):
                for g in range(groups):
                    sl = pl.ds(g * _LANES, _LANES)
                    plsc.addupdate(rows_v.at[r, sl], pos_v[r, sl])
                return carry
            lax.fori_loop(r_lo, r_hi, row_add, 0)

        for b in range(B):
            idx_d[b].wait()
            gd = pltpu.async_copy(tok_hbm.at[idx_v.at[b]], rows_v, sem_g)
            if b == 0:
                pos_d.wait()
            gd.wait()
            base = b * T + t0
            add_rows(0, half)
            w1 = pltpu.async_copy(rows_v.at[pl.ds(0, half)],
                                  out_hbm.at[pl.ds(base, half)], sem_w)
            add_rows(half, tpw)
            w2 = pltpu.async_copy(rows_v.at[pl.ds(half, half)],
                                  out_hbm.at[pl.ds(base + half, half)],
                                  sem_w)
            w1.wait()
            w2.wait()

    return emb


def kernel(x, tok_table, pos_table):
    B, T = x.shape
    V, D = tok_table.shape
    emb = _make_kernel(B, T, D, V)
    out = emb(x.astype(jnp.int32), tok_table, pos_table)
    return out.reshape(B, T, D)


# speedup vs baseline: 1.0555x; 1.0555x over previous
import functools

import jax
import jax.numpy as jnp
from jax import lax
from jax.experimental import pallas as pl
from jax.experimental.pallas import tpu as pltpu
from jax.experimental.pallas import tpu_sc as plsc

_NUM_CORES = 2
_NUM_SUBCORES = 16
_NW = _NUM_CORES * _NUM_SUBCORES
_LANES = 16


@functools.lru_cache(maxsize=None)
def _make_kernel(B, T, D, V):
    assert T % _NW == 0 and D % _LANES == 0
    tpw = T // _NW
    half = tpw // 2
    groups = D // _LANES

    mesh = plsc.VectorSubcoreMesh(core_axis_name="c", subcore_axis_name="s")

    @functools.partial(
        pl.kernel,
        mesh=mesh,
        out_type=jax.ShapeDtypeStruct((B * T, D), jnp.float32),
        scratch_types=[
            pltpu.VMEM((B, tpw), jnp.int32),
            pltpu.VMEM((tpw, D), jnp.float32),
            pltpu.VMEM((tpw, D), jnp.float32),
            pltpu.SemaphoreType.DMA,
            pltpu.SemaphoreType.DMA,
            pltpu.SemaphoreType.DMA,
            pltpu.SemaphoreType.DMA,
            pltpu.SemaphoreType.DMA,
            pltpu.SemaphoreType.DMA,
        ],
    )
    def emb(x_hbm, tok_hbm, pos_hbm, out_hbm, idx_v, rows_v, pos_v,
            sem_i, sem_p, sem_g1, sem_g2, sem_w1, sem_w2):
        wid = lax.axis_index("s") * _NUM_CORES + lax.axis_index("c")
        t0 = wid * tpw

        idx_d = [pltpu.async_copy(x_hbm.at[b, pl.ds(t0, tpw)],
                                  idx_v.at[b], sem_i) for b in range(B)]
        pos_d = pltpu.async_copy(pos_hbm.at[pl.ds(t0, tpw)], pos_v, sem_p)

        def add_rows(r_lo, r_hi):
            def row_add(r, carry):
                for g in range(groups):
                    sl = pl.ds(g * _LANES, _LANES)
                    plsc.addupdate(rows_v.at[r, sl], pos_v[r, sl])
                return carry
            lax.fori_loop(r_lo, r_hi, row_add, 0)

        w1 = w2 = None
        for b in range(B):
            idx_d[b].wait()
            if w1 is not None:
                w1.wait()
            g1 = pltpu.async_copy(
                tok_hbm.at[idx_v.at[b, pl.ds(0, half)]],
                rows_v.at[pl.ds(0, half)], sem_g1)
            if w2 is not None:
                w2.wait()
            g2 = pltpu.async_copy(
                tok_hbm.at[idx_v.at[b, pl.ds(half, half)]],
                rows_v.at[pl.ds(half, half)], sem_g2)
            if b == 0:
                pos_d.wait()
            base = b * T + t0
            g1.wait()
            add_rows(0, half)
            w1 = pltpu.async_copy(rows_v.at[pl.ds(0, half)],
                                  out_hbm.at[pl.ds(base, half)], sem_w1)
            g2.wait()
            add_rows(half, tpw)
            w2 = pltpu.async_copy(rows_v.at[pl.ds(half, half)],
                                  out_hbm.at[pl.ds(base + half, half)],
                                  sem_w2)
        w1.wait()
        w2.wait()

    return emb


def kernel(x, tok_table, pos_table):
    B, T = x.shape
    V, D = tok_table.shape
    emb = _make_kernel(B, T, D, V)
    out = emb(x.astype(jnp.int32), tok_table, pos_table)
    return out.reshape(B, T, D)
